# baseline (speedup 1.0000x reference)
import functools

import jax
import jax.numpy as jnp
from jax import lax
from jax.experimental import pallas as pl
from jax.experimental.pallas import tpu as pltpu

_RES = 64
_TL = 128


def _ingp_kernel(xt_ref, tbl_hbm, out_ref, tbl_vmem, idxv, wv, idxs, wts,
                 tsem, ssem, *, tmask):
    j = pl.program_id(1)

    @pl.when(j == 0)
    def _start_load():
        pltpu.make_async_copy(tbl_hbm, tbl_vmem, tsem).start()

    xs = xt_ref[...] * jnp.float32(_RES)
    ii = xs.astype(jnp.int32)
    fr = xs - ii.astype(jnp.float32)
    i0 = ii[0:1, :]
    i1 = ii[1:2, :]
    i2 = ii[2:3, :]
    a1 = (i1 << 4) + (i1 << 1) + i1
    a2 = (i2 << 8) + (i2 << 7) + (i2 << 2) + i2
    a1h = a1 + 19
    a2h = a2 + 389
    f0 = fr[0:1, :]
    f1 = fr[1:2, :]
    f2 = fr[2:3, :]
    g0 = 1.0 - f0
    g1 = 1.0 - f1
    g2 = 1.0 - f2
    pxy = (i0 ^ a1, i0 ^ a1h, (i0 + 1) ^ a1, (i0 + 1) ^ a1h)
    wxy = (g0 * g1, g0 * f1, f0 * g1, f0 * f1)
    c = 0
    for q in range(4):
        for az, wz in ((a2, g2), (a2h, f2)):
            idxv[c:c + 1, :] = (pxy[q] ^ az) & tmask
            wv[c:c + 1, :] = wxy[q] * wz
            c += 1
    pltpu.make_async_copy(idxv, idxs, ssem).start()
    pltpu.make_async_copy(wv, wts, ssem).start()
    pltpu.make_async_copy(idxv, idxs, ssem).wait()
    pltpu.make_async_copy(wv, wts, ssem).wait()

    @pl.when(j == 0)
    def _finish_load():
        pltpu.make_async_copy(tbl_vmem, tbl_vmem, tsem).wait()

    lane = lax.broadcasted_iota(jnp.int32, (1, 128), 1)

    def quad_body(qv, acc):
        placed = []
        for u in range(4):
            p = qv * 4 + u
            terms = []
            for c in range(8):
                idx = idxs[c, p]
                w = wts[c, p]
                off = pl.multiple_of((idx >> 7) * 128, 128)
                slab = tbl_vmem[:, pl.ds(off, 128)]
                wm = jnp.where(lane == (idx & 127), w, jnp.float32(0.0))
                terms.append(slab * wm)
            pacc = (((terms[0] + terms[1]) + (terms[2] + terms[3]))
                    + ((terms[4] + terms[5]) + (terms[6] + terms[7])))
            tot = jnp.sum(pacc, axis=1, keepdims=True)
            tot = jax.lax.broadcast_in_dim(tot, (32, 128), (0, 1))
            placed.append(jnp.where(lane == p, tot, jnp.float32(0.0)))
        return acc + ((placed[0] + placed[1]) + (placed[2] + placed[3]))

    acc = lax.fori_loop(0, _TL // 4, quad_body,
                        jnp.zeros((32, 128), jnp.float32))
    out_ref[...] = acc


def kernel(x, table):
    b, d = x.shape
    t, f = table.shape
    assert d == 3 and f == 32
    assert t & (t - 1) == 0 and b % (2 * _TL) == 0

    x_t = jnp.transpose(x)
    tbl_t = jnp.transpose(table)
    pb = b // 2
    ntiles = pb // _TL
    kern = functools.partial(_ingp_kernel, tmask=t - 1)

    out_t = pl.pallas_call(
        kern,
        out_shape=jax.ShapeDtypeStruct((f, b), x.dtype),
        grid=(2, ntiles),
        in_specs=[
            pl.BlockSpec((3, _TL), lambda i, j: (0, i * ntiles + j)),
            pl.BlockSpec(memory_space=pl.ANY),
        ],
        out_specs=pl.BlockSpec((f, _TL), lambda i, j: (0, i * ntiles + j)),
        scratch_shapes=[
            pltpu.VMEM((f, t), jnp.float32),
            pltpu.VMEM((8, _TL), jnp.int32),
            pltpu.VMEM((8, _TL), jnp.float32),
            pltpu.SMEM((8, _TL), jnp.int32),
            pltpu.SMEM((8, _TL), jnp.float32),
            pltpu.SemaphoreType.DMA,
            pltpu.SemaphoreType.DMA,
        ],
        compiler_params=pltpu.CompilerParams(
            dimension_semantics=("parallel", "arbitrary"),
            vmem_limit_bytes=40 << 20,
            disable_bounds_checks=True,
        ),
    )(x_t, tbl_t)
    return jnp.transpose(out_t)


# speedup vs baseline: 1.1321x; 1.1321x over previous
import functools

import jax
import jax.numpy as jnp
from jax import lax
from jax.experimental import pallas as pl
from jax.experimental.pallas import tpu as pltpu

_RES = 64
_TL = 128


def _ingp_kernel(xt_ref, tbl_hbm, out_ref, tbl_vmem, idxv, wv, idxs, wts,
                 tsem, ssem, *, tmask):
    j = pl.program_id(1)

    @pl.when(j == 0)
    def _start_load():
        pltpu.make_async_copy(tbl_hbm, tbl_vmem, tsem).start()

    xs = xt_ref[...] * jnp.float32(_RES)
    ii = xs.astype(jnp.int32)
    fr = xs - ii.astype(jnp.float32)
    i0 = ii[0:1, :]
    i1 = ii[1:2, :]
    i2 = ii[2:3, :]
    a1 = (i1 << 4) + (i1 << 1) + i1
    a2 = (i2 << 8) + (i2 << 7) + (i2 << 2) + i2
    a1h = a1 + 19
    a2h = a2 + 389
    f0 = fr[0:1, :]
    f1 = fr[1:2, :]
    f2 = fr[2:3, :]
    g0 = 1.0 - f0
    g1 = 1.0 - f1
    g2 = 1.0 - f2
    pxy = (i0 ^ a1, i0 ^ a1h, (i0 + 1) ^ a1, (i0 + 1) ^ a1h)
    wxy = (g0 * g1, g0 * f1, f0 * g1, f0 * f1)
    c = 0
    for q in range(4):
        for az, wz in ((a2, g2), (a2h, f2)):
            idxv[c:c + 1, :] = (pxy[q] ^ az) & tmask
            wv[c:c + 1, :] = wxy[q] * wz
            c += 1
    pltpu.make_async_copy(idxv, idxs, ssem).start()
    pltpu.make_async_copy(wv, wts, ssem).start()
    pltpu.make_async_copy(idxv, idxs, ssem).wait()
    pltpu.make_async_copy(wv, wts, ssem).wait()

    @pl.when(j == 0)
    def _finish_load():
        pltpu.make_async_copy(tbl_vmem, tbl_vmem, tsem).wait()

    lane = lax.broadcasted_iota(jnp.int32, (1, 128), 1)

    def quad_body(qv, acc):
        placed = []
        for u in range(8):
            p = qv * 8 + u
            terms = []
            for c in range(8):
                idx = idxs[c, p]
                w = wts[c, p]
                off = pl.multiple_of((idx >> 7) * 128, 128)
                slab = tbl_vmem[:, pl.ds(off, 128)]
                wm = jnp.where(lane == (idx & 127), w, jnp.float32(0.0))
                terms.append(slab * wm)
            pacc = (((terms[0] + terms[1]) + (terms[2] + terms[3]))
                    + ((terms[4] + terms[5]) + (terms[6] + terms[7])))
            tot = jnp.sum(pacc, axis=1, keepdims=True)
            tot = jax.lax.broadcast_in_dim(tot, (32, 128), (0, 1))
            placed.append(jnp.where(lane == p, tot, jnp.float32(0.0)))
        q01 = placed[0] + placed[1]
        q23 = placed[2] + placed[3]
        q45 = placed[4] + placed[5]
        q67 = placed[6] + placed[7]
        return acc + ((q01 + q23) + (q45 + q67))

    acc = lax.fori_loop(0, _TL // 8, quad_body,
                        jnp.zeros((32, 128), jnp.float32))
    out_ref[...] = acc


def kernel(x, table):
    b, d = x.shape
    t, f = table.shape
    assert d == 3 and f == 32
    assert t & (t - 1) == 0 and b % (2 * _TL) == 0

    x_t = jnp.transpose(x)
    tbl_t = jnp.transpose(table)
    pb = b // 2
    ntiles = pb // _TL
    kern = functools.partial(_ingp_kernel, tmask=t - 1)

    out_t = pl.pallas_call(
        kern,
        out_shape=jax.ShapeDtypeStruct((f, b), x.dtype),
        grid=(2, ntiles),
        in_specs=[
            pl.BlockSpec((3, _TL), lambda i, j: (0, i * ntiles + j)),
            pl.BlockSpec(memory_space=pl.ANY),
        ],
        out_specs=pl.BlockSpec((f, _TL), lambda i, j: (0, i * ntiles + j)),
        scratch_shapes=[
            pltpu.VMEM((f, t), jnp.float32),
            pltpu.VMEM((8, _TL), jnp.int32),
            pltpu.VMEM((8, _TL), jnp.float32),
            pltpu.SMEM((8, _TL), jnp.int32),
            pltpu.SMEM((8, _TL), jnp.float32),
            pltpu.SemaphoreType.DMA,
            pltpu.SemaphoreType.DMA,
        ],
        compiler_params=pltpu.CompilerParams(
            dimension_semantics=("parallel", "arbitrary"),
            vmem_limit_bytes=40 << 20,
            disable_bounds_checks=True,
        ),
    )(x_t, tbl_t)
    return jnp.transpose(out_t)


# speedup vs baseline: 1.1652x; 1.0292x over previous
import functools

import jax
import jax.numpy as jnp
from jax import lax
from jax.experimental import pallas as pl
from jax.experimental.pallas import tpu as pltpu

_RES = 64
_TL = 128


def _ingp_kernel(xt_ref, tbl_hbm, out_ref, tbl_vmem, idxv, wv, idxs, wts,
                 tsem, ssem, *, tmask):
    j = pl.program_id(1)

    @pl.when(j == 0)
    def _start_load():
        pltpu.make_async_copy(tbl_hbm, tbl_vmem, tsem).start()

    xs = xt_ref[...] * jnp.float32(_RES)
    ii = xs.astype(jnp.int32)
    fr = xs - ii.astype(jnp.float32)
    i0 = ii[0:1, :]
    i1 = ii[1:2, :]
    i2 = ii[2:3, :]
    a1 = (i1 << 4) + (i1 << 1) + i1
    a2 = (i2 << 8) + (i2 << 7) + (i2 << 2) + i2
    a1h = a1 + 19
    a2h = a2 + 389
    f0 = fr[0:1, :]
    f1 = fr[1:2, :]
    f2 = fr[2:3, :]
    g0 = 1.0 - f0
    g1 = 1.0 - f1
    g2 = 1.0 - f2
    pxy = (i0 ^ a1, i0 ^ a1h, (i0 + 1) ^ a1, (i0 + 1) ^ a1h)
    wxy = (g0 * g1, g0 * f1, f0 * g1, f0 * f1)
    c = 0
    for q in range(4):
        for az, wz in ((a2, g2), (a2h, f2)):
            idxv[c:c + 1, :] = (pxy[q] ^ az) & tmask
            wv[c:c + 1, :] = wxy[q] * wz
            c += 1
    pltpu.make_async_copy(idxv, idxs, ssem).start()
    pltpu.make_async_copy(wv, wts, ssem).start()
    pltpu.make_async_copy(idxv, idxs, ssem).wait()
    pltpu.make_async_copy(wv, wts, ssem).wait()

    @pl.when(j == 0)
    def _finish_load():
        pltpu.make_async_copy(tbl_vmem, tbl_vmem, tsem).wait()

    lane = lax.broadcasted_iota(jnp.int32, (1, 128), 1)

    def quad_body(qv, acc):
        placed = []
        for u in range(16):
            p = qv * 16 + u
            terms = []
            for c in range(8):
                idx = idxs[c, p]
                w = wts[c, p]
                off = pl.multiple_of((idx >> 7) * 128, 128)
                slab = tbl_vmem[:, pl.ds(off, 128)]
                wm = jnp.where(lane == (idx & 127), w, jnp.float32(0.0))
                terms.append(slab * wm)
            pacc = (((terms[0] + terms[1]) + (terms[2] + terms[3]))
                    + ((terms[4] + terms[5]) + (terms[6] + terms[7])))
            tot = jnp.sum(pacc, axis=1, keepdims=True)
            tot = jax.lax.broadcast_in_dim(tot, (32, 128), (0, 1))
            placed.append(jnp.where(lane == p, tot, jnp.float32(0.0)))
        while len(placed) > 1:
            placed = [placed[k] + placed[k + 1] for k in range(0, len(placed), 2)]
        return acc + placed[0]

    acc = lax.fori_loop(0, _TL // 16, quad_body,
                        jnp.zeros((32, 128), jnp.float32))
    out_ref[...] = acc


def kernel(x, table):
    b, d = x.shape
    t, f = table.shape
    assert d == 3 and f == 32
    assert t & (t - 1) == 0 and b % (2 * _TL) == 0

    x_t = jnp.transpose(x)
    tbl_t = jnp.transpose(table)
    pb = b // 2
    ntiles = pb // _TL
    kern = functools.partial(_ingp_kernel, tmask=t - 1)

    out_t = pl.pallas_call(
        kern,
        out_shape=jax.ShapeDtypeStruct((f, b), x.dtype),
        grid=(2, ntiles),
        in_specs=[
            pl.BlockSpec((3, _TL), lambda i, j: (0, i * ntiles + j)),
            pl.BlockSpec(memory_space=pl.ANY),
        ],
        out_specs=pl.BlockSpec((f, _TL), lambda i, j: (0, i * ntiles + j)),
        scratch_shapes=[
            pltpu.VMEM((f, t), jnp.float32),
            pltpu.VMEM((8, _TL), jnp.int32),
            pltpu.VMEM((8, _TL), jnp.float32),
            pltpu.SMEM((8, _TL), jnp.int32),
            pltpu.SMEM((8, _TL), jnp.float32),
            pltpu.SemaphoreType.DMA,
            pltpu.SemaphoreType.DMA,
        ],
        compiler_params=pltpu.CompilerParams(
            dimension_semantics=("parallel", "arbitrary"),
            vmem_limit_bytes=40 << 20,
            disable_bounds_checks=True,
        ),
    )(x_t, tbl_t)
    return jnp.transpose(out_t)


# speedup vs baseline: 1.2114x; 1.0397x over previous
import functools

import jax
import jax.numpy as jnp
from jax import lax
from jax.experimental import pallas as pl
from jax.experimental.pallas import tpu as pltpu

_RES = 64
_TL = 128


def _ingp_kernel(xt_ref, tbl_hbm, out_ref, tbl_vmem, idxv, wv, idxs, wts,
                 tsem, ssem, *, tmask):
    j = pl.program_id(1)

    nl = tbl_vmem.shape[1]

    @pl.when(j == 0)
    def _start_load():
        for k in range(4):
            sl = pl.ds(k * (nl // 4), nl // 4)
            pltpu.make_async_copy(tbl_hbm.at[:, sl], tbl_vmem.at[:, sl],
                                  tsem).start()

    xs = xt_ref[...] * jnp.float32(_RES)
    ii = xs.astype(jnp.int32)
    fr = xs - ii.astype(jnp.float32)
    i0 = ii[0:1, :]
    i1 = ii[1:2, :]
    i2 = ii[2:3, :]
    a1 = (i1 << 4) + (i1 << 1) + i1
    a2 = (i2 << 8) + (i2 << 7) + (i2 << 2) + i2
    a1h = a1 + 19
    a2h = a2 + 389
    f0 = fr[0:1, :]
    f1 = fr[1:2, :]
    f2 = fr[2:3, :]
    g0 = 1.0 - f0
    g1 = 1.0 - f1
    g2 = 1.0 - f2
    pxy = (i0 ^ a1, i0 ^ a1h, (i0 + 1) ^ a1, (i0 + 1) ^ a1h)
    wxy = (g0 * g1, g0 * f1, f0 * g1, f0 * f1)
    c = 0
    for q in range(4):
        for az, wz in ((a2, g2), (a2h, f2)):
            idxv[c:c + 1, :] = (pxy[q] ^ az) & tmask
            wv[c:c + 1, :] = wxy[q] * wz
            c += 1
    pltpu.make_async_copy(idxv, idxs, ssem).start()
    pltpu.make_async_copy(wv, wts, ssem).start()
    pltpu.make_async_copy(idxv, idxs, ssem).wait()
    pltpu.make_async_copy(wv, wts, ssem).wait()

    @pl.when(j == 0)
    def _finish_load():
        for k in range(4):
            sl = pl.ds(k * (nl // 4), nl // 4)
            pltpu.make_async_copy(tbl_vmem.at[:, sl], tbl_vmem.at[:, sl],
                                  tsem).wait()

    lane = lax.broadcasted_iota(jnp.int32, (1, 128), 1)

    def quad_body(qv, acc):
        placed = []
        for u in range(16):
            p = qv * 16 + u
            terms = []
            for c in range(8):
                idx = idxs[c, p]
                w = wts[c, p]
                off = pl.multiple_of((idx >> 7) * 128, 128)
                slab = tbl_vmem[:, pl.ds(off, 128)]
                wm = jnp.where(lane == (idx & 127), w, jnp.float32(0.0))
                terms.append(slab * wm)
            pacc = (((terms[0] + terms[1]) + (terms[2] + terms[3]))
                    + ((terms[4] + terms[5]) + (terms[6] + terms[7])))
            tot = jnp.sum(pacc, axis=1, keepdims=True)
            tot = jax.lax.broadcast_in_dim(tot, (32, 128), (0, 1))
            placed.append(jnp.where(lane == p, tot, jnp.float32(0.0)))
        while len(placed) > 1:
            placed = [placed[k] + placed[k + 1] for k in range(0, len(placed), 2)]
        return acc + placed[0]

    acc = lax.fori_loop(0, _TL // 16, quad_body,
                        jnp.zeros((32, 128), jnp.float32))
    out_ref[...] = acc


def kernel(x, table):
    b, d = x.shape
    t, f = table.shape
    assert d == 3 and f == 32
    assert t & (t - 1) == 0 and b % (2 * _TL) == 0

    x_t = jnp.transpose(x)
    tbl_t = jnp.transpose(table)
    pb = b // 2
    ntiles = pb // _TL
    kern = functools.partial(_ingp_kernel, tmask=t - 1)

    out_t = pl.pallas_call(
        kern,
        out_shape=jax.ShapeDtypeStruct((f, b), x.dtype),
        grid=(2, ntiles),
        in_specs=[
            pl.BlockSpec((3, _TL), lambda i, j: (0, i * ntiles + j)),
            pl.BlockSpec(memory_space=pl.ANY),
        ],
        out_specs=pl.BlockSpec((f, _TL), lambda i, j: (0, i * ntiles + j)),
        scratch_shapes=[
            pltpu.VMEM((f, t), jnp.float32),
            pltpu.VMEM((8, _TL), jnp.int32),
            pltpu.VMEM((8, _TL), jnp.float32),
            pltpu.SMEM((8, _TL), jnp.int32),
            pltpu.SMEM((8, _TL), jnp.float32),
            pltpu.SemaphoreType.DMA,
            pltpu.SemaphoreType.DMA,
        ],
        compiler_params=pltpu.CompilerParams(
            dimension_semantics=("parallel", "arbitrary"),
            vmem_limit_bytes=40 << 20,
            disable_bounds_checks=True,
        ),
    )(x_t, tbl_t)
    return jnp.transpose(out_t)


# speedup vs baseline: 1.2752x; 1.0526x over previous
import functools

import jax
import jax.numpy as jnp
from jax import lax
from jax.experimental import pallas as pl
from jax.experimental.pallas import tpu as pltpu

_RES = 64
_TL = 128


def _ingp_kernel(xt_ref, tbl_hbm, out_ref, tbl_vmem, idxv, lnv, wv, idxs, lns,
                 wts, tsem, ssem, *, tmask):
    j = pl.program_id(1)

    nl = tbl_vmem.shape[1]

    @pl.when(j == 0)
    def _start_load():
        for k in range(8):
            sl = pl.ds(k * (nl // 8), nl // 8)
            pltpu.make_async_copy(tbl_hbm.at[:, sl], tbl_vmem.at[:, sl],
                                  tsem).start()

    xs = xt_ref[...] * jnp.float32(_RES)
    ii = xs.astype(jnp.int32)
    fr = xs - ii.astype(jnp.float32)
    i0 = ii[0:1, :]
    i1 = ii[1:2, :]
    i2 = ii[2:3, :]
    a1 = (i1 << 4) + (i1 << 1) + i1
    a2 = (i2 << 8) + (i2 << 7) + (i2 << 2) + i2
    a1h = a1 + 19
    a2h = a2 + 389
    f0 = fr[0:1, :]
    f1 = fr[1:2, :]
    f2 = fr[2:3, :]
    g0 = 1.0 - f0
    g1 = 1.0 - f1
    g2 = 1.0 - f2
    pxy = (i0 ^ a1, i0 ^ a1h, (i0 + 1) ^ a1, (i0 + 1) ^ a1h)
    wxy = (g0 * g1, g0 * f1, f0 * g1, f0 * f1)
    c = 0
    for q in range(4):
        for az, wz in ((a2, g2), (a2h, f2)):
            hidx = (pxy[q] ^ az) & tmask
            idxv[c:c + 1, :] = (hidx >> 7) << 7
            lnv[c:c + 1, :] = hidx & 127
            wv[c:c + 1, :] = wxy[q] * wz
            c += 1
    pltpu.make_async_copy(idxv, idxs, ssem).start()
    pltpu.make_async_copy(lnv, lns, ssem).start()
    pltpu.make_async_copy(wv, wts, ssem).start()
    pltpu.make_async_copy(idxv, idxs, ssem).wait()
    pltpu.make_async_copy(lnv, lns, ssem).wait()
    pltpu.make_async_copy(wv, wts, ssem).wait()

    @pl.when(j == 0)
    def _finish_load():
        for k in range(8):
            sl = pl.ds(k * (nl // 8), nl // 8)
            pltpu.make_async_copy(tbl_vmem.at[:, sl], tbl_vmem.at[:, sl],
                                  tsem).wait()

    lane = lax.broadcasted_iota(jnp.int32, (1, 128), 1)

    def quad_body(qv, acc):
        placed = []
        for u in range(16):
            p = qv * 16 + u
            terms = []
            for c in range(8):
                off = pl.multiple_of(idxs[c, p], 128)
                w = wts[c, p]
                slab = tbl_vmem[:, pl.ds(off, 128)]
                wm = jnp.where(lane == lns[c, p], w, jnp.float32(0.0))
                terms.append(slab * wm)
            pacc = (((terms[0] + terms[1]) + (terms[2] + terms[3]))
                    + ((terms[4] + terms[5]) + (terms[6] + terms[7])))
            tot = jnp.sum(pacc, axis=1, keepdims=True)
            tot = jax.lax.broadcast_in_dim(tot, (32, 128), (0, 1))
            placed.append(jnp.where(lane == p, tot, jnp.float32(0.0)))
        while len(placed) > 1:
            placed = [placed[k] + placed[k + 1] for k in range(0, len(placed), 2)]
        return acc + placed[0]

    acc = lax.fori_loop(0, _TL // 16, quad_body,
                        jnp.zeros((32, 128), jnp.float32))
    out_ref[...] = acc


def kernel(x, table):
    b, d = x.shape
    t, f = table.shape
    assert d == 3 and f == 32
    assert t & (t - 1) == 0 and b % (2 * _TL) == 0

    x_t = jnp.transpose(x)
    tbl_t = jnp.transpose(table)
    pb = b // 2
    ntiles = pb // _TL
    kern = functools.partial(_ingp_kernel, tmask=t - 1)

    out_t = pl.pallas_call(
        kern,
        out_shape=jax.ShapeDtypeStruct((f, b), x.dtype),
        grid=(2, ntiles),
        in_specs=[
            pl.BlockSpec((3, _TL), lambda i, j: (0, i * ntiles + j)),
            pl.BlockSpec(memory_space=pl.ANY),
        ],
        out_specs=pl.BlockSpec((f, _TL), lambda i, j: (0, i * ntiles + j)),
        scratch_shapes=[
            pltpu.VMEM((f, t), jnp.float32),
            pltpu.VMEM((8, _TL), jnp.int32),
            pltpu.VMEM((8, _TL), jnp.int32),
            pltpu.VMEM((8, _TL), jnp.float32),
            pltpu.SMEM((8, _TL), jnp.int32),
            pltpu.SMEM((8, _TL), jnp.int32),
            pltpu.SMEM((8, _TL), jnp.float32),
            pltpu.SemaphoreType.DMA,
            pltpu.SemaphoreType.DMA,
        ],
        compiler_params=pltpu.CompilerParams(
            dimension_semantics=("parallel", "arbitrary"),
            vmem_limit_bytes=40 << 20,
            disable_bounds_checks=True,
        ),
    )(x_t, tbl_t)
    return jnp.transpose(out_t)


# speedup vs baseline: 1.3221x; 1.0367x over previous
import functools

import jax
import jax.numpy as jnp
from jax import lax
from jax.experimental import pallas as pl
from jax.experimental.pallas import tpu as pltpu

_RES = 64
_TL = 128


def _ingp_kernel(xt_ref, tbl_hbm, out_ref, tbl_vmem, idxv, lnv, wv, idxs, lns,
                 wts, tsem, ssem, *, tmask):
    j = pl.program_id(1)

    nl = tbl_vmem.shape[1]

    @pl.when(j == 0)
    def _start_load():
        for k in range(8):
            sl = pl.ds(k * (nl // 8), nl // 8)
            pltpu.make_async_copy(tbl_hbm.at[:, sl], tbl_vmem.at[:, sl],
                                  tsem).start()

    xs = xt_ref[...] * jnp.float32(_RES)
    ii = xs.astype(jnp.int32)
    fr = xs - ii.astype(jnp.float32)
    i0 = ii[0:1, :]
    i1 = ii[1:2, :]
    i2 = ii[2:3, :]
    a1 = (i1 << 4) + (i1 << 1) + i1
    a2 = (i2 << 8) + (i2 << 7) + (i2 << 2) + i2
    a1h = a1 + 19
    a2h = a2 + 389
    f0 = fr[0:1, :]
    f1 = fr[1:2, :]
    f2 = fr[2:3, :]
    g0 = 1.0 - f0
    g1 = 1.0 - f1
    g2 = 1.0 - f2
    pxy = (i0 ^ a1, i0 ^ a1h, (i0 + 1) ^ a1, (i0 + 1) ^ a1h)
    wxy = (g0 * g1, g0 * f1, f0 * g1, f0 * f1)
    c = 0
    for q in range(4):
        for az, wz in ((a2, g2), (a2h, f2)):
            hidx = (pxy[q] ^ az) & tmask
            idxv[c:c + 1, :] = (hidx >> 7) << 7
            lnv[c:c + 1, :] = hidx & 127
            wv[c:c + 1, :] = wxy[q] * wz
            c += 1
    pltpu.make_async_copy(idxv, idxs, ssem).start()
    pltpu.make_async_copy(lnv, lns, ssem).start()
    pltpu.make_async_copy(wv, wts, ssem).start()
    pltpu.make_async_copy(idxv, idxs, ssem).wait()
    pltpu.make_async_copy(lnv, lns, ssem).wait()
    pltpu.make_async_copy(wv, wts, ssem).wait()

    @pl.when(j == 0)
    def _finish_load():
        for k in range(8):
            sl = pl.ds(k * (nl // 8), nl // 8)
            pltpu.make_async_copy(tbl_vmem.at[:, sl], tbl_vmem.at[:, sl],
                                  tsem).wait()

    lane = lax.broadcasted_iota(jnp.int32, (1, 128), 1)

    def quad_body(qv, acc):
        placed = []
        for u in range(32):
            p = qv * 32 + u
            terms = []
            for c in range(8):
                off = pl.multiple_of(idxs[c, p], 128)
                w = wts[c, p]
                slab = tbl_vmem[:, pl.ds(off, 128)]
                wm = jnp.where(lane == lns[c, p], w, jnp.float32(0.0))
                terms.append(slab * wm)
            pacc = (((terms[0] + terms[1]) + (terms[2] + terms[3]))
                    + ((terms[4] + terms[5]) + (terms[6] + terms[7])))
            tot = jnp.sum(pacc, axis=1, keepdims=True)
            tot = jax.lax.broadcast_in_dim(tot, (32, 128), (0, 1))
            placed.append(jnp.where(lane == p, tot, jnp.float32(0.0)))
        while len(placed) > 1:
            placed = [placed[k] + placed[k + 1] for k in range(0, len(placed), 2)]
        return acc + placed[0]

    acc = lax.fori_loop(0, _TL // 32, quad_body,
                        jnp.zeros((32, 128), jnp.float32))
    out_ref[...] = acc


def kernel(x, table):
    b, d = x.shape
    t, f = table.shape
    assert d == 3 and f == 32
    assert t & (t - 1) == 0 and b % (2 * _TL) == 0

    x_t = jnp.transpose(x)
    tbl_t = jnp.transpose(table)
    pb = b // 2
    ntiles = pb // _TL
    kern = functools.partial(_ingp_kernel, tmask=t - 1)

    out_t = pl.pallas_call(
        kern,
        out_shape=jax.ShapeDtypeStruct((f, b), x.dtype),
        grid=(2, ntiles),
        in_specs=[
            pl.BlockSpec((3, _TL), lambda i, j: (0, i * ntiles + j)),
            pl.BlockSpec(memory_space=pl.ANY),
        ],
        out_specs=pl.BlockSpec((f, _TL), lambda i, j: (0, i * ntiles + j)),
        scratch_shapes=[
            pltpu.VMEM((f, t), jnp.float32),
            pltpu.VMEM((8, _TL), jnp.int32),
            pltpu.VMEM((8, _TL), jnp.int32),
            pltpu.VMEM((8, _TL), jnp.float32),
            pltpu.SMEM((8, _TL), jnp.int32),
            pltpu.SMEM((8, _TL), jnp.int32),
            pltpu.SMEM((8, _TL), jnp.float32),
            pltpu.SemaphoreType.DMA,
            pltpu.SemaphoreType.DMA,
        ],
        compiler_params=pltpu.CompilerParams(
            dimension_semantics=("parallel", "arbitrary"),
            vmem_limit_bytes=40 << 20,
            disable_bounds_checks=True,
        ),
    )(x_t, tbl_t)
    return jnp.transpose(out_t)


# speedup vs baseline: 1.3751x; 1.0401x over previous
import functools

import jax
import jax.numpy as jnp
from jax import lax
from jax.experimental import pallas as pl
from jax.experimental.pallas import tpu as pltpu

_RES = 64
_TL = 128


def _ingp_kernel(xt_ref, tbl_hbm, out_ref, tbl_vmem, idxv, lnv, wv, idxs, lns,
                 wts, tsem, ssem, *, tmask):
    j = pl.program_id(1)

    nl = tbl_vmem.shape[1]

    @pl.when(j == 0)
    def _start_load():
        for k in range(8):
            sl = pl.ds(k * (nl // 8), nl // 8)
            pltpu.make_async_copy(tbl_hbm.at[:, sl], tbl_vmem.at[:, sl],
                                  tsem).start()

    xs = xt_ref[...] * jnp.float32(_RES)
    ii = xs.astype(jnp.int32)
    fr = xs - ii.astype(jnp.float32)
    i0 = ii[0:1, :]
    i1 = ii[1:2, :]
    i2 = ii[2:3, :]
    a1 = (i1 << 4) + (i1 << 1) + i1
    a2 = (i2 << 8) + (i2 << 7) + (i2 << 2) + i2
    a1h = a1 + 19
    a2h = a2 + 389
    f0 = fr[0:1, :]
    f1 = fr[1:2, :]
    f2 = fr[2:3, :]
    g0 = 1.0 - f0
    g1 = 1.0 - f1
    g2 = 1.0 - f2
    pxy = (i0 ^ a1, i0 ^ a1h, (i0 + 1) ^ a1, (i0 + 1) ^ a1h)
    wxy = (g0 * g1, g0 * f1, f0 * g1, f0 * f1)
    c = 0
    for q in range(4):
        for az, wz in ((a2, g2), (a2h, f2)):
            hidx = (pxy[q] ^ az) & tmask
            idxv[c:c + 1, :] = (hidx >> 7) << 7
            lnv[c:c + 1, :] = hidx & 127
            wv[c:c + 1, :] = wxy[q] * wz
            c += 1
    pltpu.make_async_copy(idxv, idxs, ssem).start()
    pltpu.make_async_copy(lnv, lns, ssem).start()
    pltpu.make_async_copy(wv, wts, ssem).start()
    pltpu.make_async_copy(idxv, idxs, ssem).wait()
    pltpu.make_async_copy(lnv, lns, ssem).wait()
    pltpu.make_async_copy(wv, wts, ssem).wait()

    @pl.when(j == 0)
    def _finish_load():
        for k in range(8):
            sl = pl.ds(k * (nl // 8), nl // 8)
            pltpu.make_async_copy(tbl_vmem.at[:, sl], tbl_vmem.at[:, sl],
                                  tsem).wait()

    lane = lax.broadcasted_iota(jnp.int32, (1, 128), 1)

    def _tile_sum():
        placed = []
        for p in range(_TL):
            terms = []
            for c in range(8):
                off = pl.multiple_of(idxs[c, p], 128)
                w = wts[c, p]
                slab = tbl_vmem[:, pl.ds(off, 128)]
                wm = jnp.where(lane == lns[c, p], w, jnp.float32(0.0))
                terms.append(slab * wm)
            pacc = (((terms[0] + terms[1]) + (terms[2] + terms[3]))
                    + ((terms[4] + terms[5]) + (terms[6] + terms[7])))
            tot = jnp.sum(pacc, axis=1, keepdims=True)
            tot = jax.lax.broadcast_in_dim(tot, (32, 128), (0, 1))
            placed.append(jnp.where(lane == p, tot, jnp.float32(0.0)))
        while len(placed) > 1:
            placed = [placed[k] + placed[k + 1] for k in range(0, len(placed), 2)]
        return placed[0]

    out_ref[...] = _tile_sum()


def kernel(x, table):
    b, d = x.shape
    t, f = table.shape
    assert d == 3 and f == 32
    assert t & (t - 1) == 0 and b % (2 * _TL) == 0

    x_t = jnp.transpose(x)
    tbl_t = jnp.transpose(table)
    pb = b // 2
    ntiles = pb // _TL
    kern = functools.partial(_ingp_kernel, tmask=t - 1)

    out_t = pl.pallas_call(
        kern,
        out_shape=jax.ShapeDtypeStruct((f, b), x.dtype),
        grid=(2, ntiles),
        in_specs=[
            pl.BlockSpec((3, _TL), lambda i, j: (0, i * ntiles + j)),
            pl.BlockSpec(memory_space=pl.ANY),
        ],
        out_specs=pl.BlockSpec((f, _TL), lambda i, j: (0, i * ntiles + j)),
        scratch_shapes=[
            pltpu.VMEM((f, t), jnp.float32),
            pltpu.VMEM((8, _TL), jnp.int32),
            pltpu.VMEM((8, _TL), jnp.int32),
            pltpu.VMEM((8, _TL), jnp.float32),
            pltpu.SMEM((8, _TL), jnp.int32),
            pltpu.SMEM((8, _TL), jnp.int32),
            pltpu.SMEM((8, _TL), jnp.float32),
            pltpu.SemaphoreType.DMA,
            pltpu.SemaphoreType.DMA,
        ],
        compiler_params=pltpu.CompilerParams(
            dimension_semantics=("parallel", "arbitrary"),
            vmem_limit_bytes=40 << 20,
            disable_bounds_checks=True,
        ),
    )(x_t, tbl_t)
    return jnp.transpose(out_t)


# speedup vs baseline: 1.3787x; 1.0027x over previous
import functools

import jax
import jax.numpy as jnp
from jax import lax
from jax.experimental import pallas as pl
from jax.experimental.pallas import tpu as pltpu

_RES = 64
_TL = 128


def _ingp_kernel(xt_ref, tbl_hbm, out_ref, tbl_vmem, idxv, lnv, wv, idxs, lns,
                 wts, tsem, ssem, *, tmask):
    j = pl.program_id(1)

    nl = tbl_vmem.shape[1]

    @pl.when(j == 0)
    def _start_load():
        for k in range(16):
            sl = pl.ds(k * (nl // 16), nl // 16)
            pltpu.make_async_copy(tbl_hbm.at[:, sl], tbl_vmem.at[:, sl],
                                  tsem).start()

    xs = xt_ref[...] * jnp.float32(_RES)
    ii = xs.astype(jnp.int32)
    fr = xs - ii.astype(jnp.float32)
    i0 = ii[0:1, :]
    i1 = ii[1:2, :]
    i2 = ii[2:3, :]
    a1 = (i1 << 4) + (i1 << 1) + i1
    a2 = (i2 << 8) + (i2 << 7) + (i2 << 2) + i2
    a1h = a1 + 19
    a2h = a2 + 389
    f0 = fr[0:1, :]
    f1 = fr[1:2, :]
    f2 = fr[2:3, :]
    g0 = 1.0 - f0
    g1 = 1.0 - f1
    g2 = 1.0 - f2
    pxy = (i0 ^ a1, i0 ^ a1h, (i0 + 1) ^ a1, (i0 + 1) ^ a1h)
    wxy = (g0 * g1, g0 * f1, f0 * g1, f0 * f1)
    c = 0
    for q in range(4):
        for az, wz in ((a2, g2), (a2h, f2)):
            hidx = (pxy[q] ^ az) & tmask
            idxv[c:c + 1, :] = (hidx >> 7) << 7
            lnv[c:c + 1, :] = hidx & 127
            wv[c:c + 1, :] = wxy[q] * wz
            c += 1
    pltpu.make_async_copy(idxv, idxs, ssem).start()
    pltpu.make_async_copy(lnv, lns, ssem).start()
    pltpu.make_async_copy(wv, wts, ssem).start()
    pltpu.make_async_copy(idxv, idxs, ssem).wait()
    pltpu.make_async_copy(lnv, lns, ssem).wait()
    pltpu.make_async_copy(wv, wts, ssem).wait()

    @pl.when(j == 0)
    def _finish_load():
        for k in range(16):
            sl = pl.ds(k * (nl // 16), nl // 16)
            pltpu.make_async_copy(tbl_vmem.at[:, sl], tbl_vmem.at[:, sl],
                                  tsem).wait()

    lane = lax.broadcasted_iota(jnp.int32, (1, 128), 1)

    def _tile_sum():
        placed = []
        for p in range(_TL):
            terms = []
            for c in range(8):
                off = pl.multiple_of(idxs[c, p], 128)
                w = wts[c, p]
                slab = tbl_vmem[:, pl.ds(off, 128)]
                wm = jnp.where(lane == lns[c, p], w, jnp.float32(0.0))
                terms.append(slab * wm)
            pacc = (((terms[0] + terms[1]) + (terms[2] + terms[3]))
                    + ((terms[4] + terms[5]) + (terms[6] + terms[7])))
            tot = jnp.sum(pacc, axis=1, keepdims=True)
            tot = jax.lax.broadcast_in_dim(tot, (32, 128), (0, 1))
            placed.append(jnp.where(lane == p, tot, jnp.float32(0.0)))
        while len(placed) > 1:
            placed = [placed[k] + placed[k + 1] for k in range(0, len(placed), 2)]
        return placed[0]

    out_ref[...] = _tile_sum()


def kernel(x, table):
    b, d = x.shape
    t, f = table.shape
    assert d == 3 and f == 32
    assert t & (t - 1) == 0 and b % (2 * _TL) == 0

    x_t = jnp.transpose(x)
    tbl_t = jnp.transpose(table)
    pb = b // 2
    ntiles = pb // _TL
    kern = functools.partial(_ingp_kernel, tmask=t - 1)

    out_t = pl.pallas_call(
        kern,
        out_shape=jax.ShapeDtypeStruct((f, b), x.dtype),
        grid=(2, ntiles),
        in_specs=[
            pl.BlockSpec((3, _TL), lambda i, j: (0, i * ntiles + j)),
            pl.BlockSpec(memory_space=pl.ANY),
        ],
        out_specs=pl.BlockSpec((f, _TL), lambda i, j: (0, i * ntiles + j)),
        scratch_shapes=[
            pltpu.VMEM((f, t), jnp.float32),
            pltpu.VMEM((8, _TL), jnp.int32),
            pltpu.VMEM((8, _TL), jnp.int32),
            pltpu.VMEM((8, _TL), jnp.float32),
            pltpu.SMEM((8, _TL), jnp.int32),
            pltpu.SMEM((8, _TL), jnp.int32),
            pltpu.SMEM((8, _TL), jnp.float32),
            pltpu.SemaphoreType.DMA,
            pltpu.SemaphoreType.DMA,
        ],
        compiler_params=pltpu.CompilerParams(
            dimension_semantics=("parallel", "arbitrary"),
            vmem_limit_bytes=40 << 20,
            disable_bounds_checks=True,
        ),
    )(x_t, tbl_t)
    return jnp.transpose(out_t)


# speedup vs baseline: 1.4200x; 1.0299x over previous
import functools

import jax
import jax.numpy as jnp
from jax import lax
from jax.experimental import pallas as pl
from jax.experimental.pallas import tpu as pltpu

_RES = 64
_TL = 128


def _ingp_kernel(xt_ref, tbl_hbm, out_ref, tbl_vmem, idxv, lnv, wv, idxs, lns,
                 wts, tsem, ssem, *, tmask):
    j = pl.program_id(1)
    nl = tbl_vmem.shape[1]

    @pl.when(j == 0)
    def _start_load():
        for k in range(16):
            sl = pl.ds(k * (nl // 16), nl // 16)
            pltpu.make_async_copy(tbl_hbm.at[:, sl], tbl_vmem.at[:, sl],
                                  tsem).start()

    @pl.when(j == 0)
    def _hash_phase():
        xs = xt_ref[...] * jnp.float32(_RES)
        ii = xs.astype(jnp.int32)
        fr = xs - ii.astype(jnp.float32)
        i0 = ii[0:1, :]
        i1 = ii[1:2, :]
        i2 = ii[2:3, :]
        a1 = (i1 << 4) + (i1 << 1) + i1
        a2 = (i2 << 8) + (i2 << 7) + (i2 << 2) + i2
        a1h = a1 + 19
        a2h = a2 + 389
        f0 = fr[0:1, :]
        f1 = fr[1:2, :]
        f2 = fr[2:3, :]
        g0 = 1.0 - f0
        g1 = 1.0 - f1
        g2 = 1.0 - f2
        pxy = (i0 ^ a1, i0 ^ a1h, (i0 + 1) ^ a1, (i0 + 1) ^ a1h)
        wxy = (g0 * g1, g0 * f1, f0 * g1, f0 * f1)
        c = 0
        for q in range(4):
            for az, wz in ((a2, g2), (a2h, f2)):
                hidx = (pxy[q] ^ az) & tmask
                idxv[c:c + 1, :] = (hidx >> 7) << 7
                lnv[c:c + 1, :] = hidx & 127
                wv[c:c + 1, :] = wxy[q] * wz
                c += 1
        pltpu.make_async_copy(idxv, idxs, ssem).start()
        pltpu.make_async_copy(lnv, lns, ssem).start()
        pltpu.make_async_copy(wv, wts, ssem).start()
        pltpu.make_async_copy(idxv, idxs, ssem).wait()
        pltpu.make_async_copy(lnv, lns, ssem).wait()
        pltpu.make_async_copy(wv, wts, ssem).wait()

    @pl.when(j == 0)
    def _finish_load():
        for k in range(16):
            sl = pl.ds(k * (nl // 16), nl // 16)
            pltpu.make_async_copy(tbl_vmem.at[:, sl], tbl_vmem.at[:, sl],
                                  tsem).wait()

    lane = lax.broadcasted_iota(jnp.int32, (1, 128), 1)
    base = j * _TL

    def _tile_sum():
        placed = []
        for p in range(_TL):
            terms = []
            for c in range(8):
                off = pl.multiple_of(idxs[c, base + p], 128)
                w = wts[c, base + p]
                slab = tbl_vmem[:, pl.ds(off, 128)]
                wm = jnp.where(lane == lns[c, base + p], w, jnp.float32(0.0))
                terms.append(slab * wm)
            pacc = (((terms[0] + terms[1]) + (terms[2] + terms[3]))
                    + ((terms[4] + terms[5]) + (terms[6] + terms[7])))
            tot = jnp.sum(pacc, axis=1, keepdims=True)
            tot = jax.lax.broadcast_in_dim(tot, (32, 128), (0, 1))
            placed.append(jnp.where(lane == p, tot, jnp.float32(0.0)))
        while len(placed) > 1:
            placed = [placed[k] + placed[k + 1] for k in range(0, len(placed), 2)]
        return placed[0]

    out_ref[...] = _tile_sum()


def kernel(x, table):
    b, d = x.shape
    t, f = table.shape
    assert d == 3 and f == 32
    assert t & (t - 1) == 0 and b % (2 * _TL) == 0

    x_t = jnp.transpose(x)
    tbl_t = jnp.transpose(table)
    pb = b // 2
    ntiles = pb // _TL
    kern = functools.partial(_ingp_kernel, tmask=t - 1)

    out_t = pl.pallas_call(
        kern,
        out_shape=jax.ShapeDtypeStruct((f, b), x.dtype),
        grid=(2, ntiles),
        in_specs=[
            pl.BlockSpec((3, pb), lambda i, j: (0, i)),
            pl.BlockSpec(memory_space=pl.ANY),
        ],
        out_specs=pl.BlockSpec((f, _TL), lambda i, j: (0, i * ntiles + j)),
        scratch_shapes=[
            pltpu.VMEM((f, t), jnp.float32),
            pltpu.VMEM((8, pb), jnp.int32),
            pltpu.VMEM((8, pb), jnp.int32),
            pltpu.VMEM((8, pb), jnp.float32),
            pltpu.SMEM((8, pb), jnp.int32),
            pltpu.SMEM((8, pb), jnp.int32),
            pltpu.SMEM((8, pb), jnp.float32),
            pltpu.SemaphoreType.DMA,
            pltpu.SemaphoreType.DMA,
        ],
        compiler_params=pltpu.CompilerParams(
            dimension_semantics=("parallel", "arbitrary"),
            vmem_limit_bytes=40 << 20,
            disable_bounds_checks=True,
        ),
    )(x_t, tbl_t)
    return jnp.transpose(out_t)


# speedup vs baseline: 1.4206x; 1.0005x over previous
import functools

import jax
import jax.numpy as jnp
from jax import lax
from jax.experimental import pallas as pl
from jax.experimental.pallas import tpu as pltpu

_RES = 64
_TL = 128


def _ingp_kernel(xt_ref, tbl_hbm, out_ref, tbl_vmem, idxv, lnv, wv, idxs, lns,
                 wts, tsem, ssem, *, tmask):
    j = pl.program_id(1)
    nl = tbl_vmem.shape[1]

    @pl.when(j == 0)
    def _start_load():
        for k in range(16):
            sl = pl.ds(k * (nl // 16), nl // 16)
            pltpu.make_async_copy(tbl_hbm.at[:, sl], tbl_vmem.at[:, sl],
                                  tsem).start()

    @pl.when(j == 0)
    def _hash_phase():
        xs = xt_ref[...] * jnp.float32(_RES)
        ii = xs.astype(jnp.int32)
        fr = xs - ii.astype(jnp.float32)
        i0 = ii[0:1, :]
        i1 = ii[1:2, :]
        i2 = ii[2:3, :]
        a1 = (i1 << 4) + (i1 << 1) + i1
        a2 = (i2 << 8) + (i2 << 7) + (i2 << 2) + i2
        a1h = a1 + 19
        a2h = a2 + 389
        f0 = fr[0:1, :]
        f1 = fr[1:2, :]
        f2 = fr[2:3, :]
        g0 = 1.0 - f0
        g1 = 1.0 - f1
        g2 = 1.0 - f2
        pxy = (i0 ^ a1, i0 ^ a1h, (i0 + 1) ^ a1, (i0 + 1) ^ a1h)
        wxy = (g0 * g1, g0 * f1, f0 * g1, f0 * f1)
        c = 0
        for q in range(4):
            for az, wz in ((a2, g2), (a2h, f2)):
                hidx = (pxy[q] ^ az) & tmask
                idxv[c:c + 1, :] = (hidx >> 7) << 7
                lnv[c:c + 1, :] = hidx & 127
                wv[c:c + 1, :] = wxy[q] * wz
                c += 1
        pltpu.make_async_copy(idxv, idxs, ssem).start()
        pltpu.make_async_copy(lnv, lns, ssem).start()
        pltpu.make_async_copy(wv, wts, ssem).start()
        pltpu.make_async_copy(idxv, idxs, ssem).wait()
        pltpu.make_async_copy(lnv, lns, ssem).wait()
        pltpu.make_async_copy(wv, wts, ssem).wait()

    @pl.when(j == 0)
    def _finish_load():
        for k in range(16):
            sl = pl.ds(k * (nl // 16), nl // 16)
            pltpu.make_async_copy(tbl_vmem.at[:, sl], tbl_vmem.at[:, sl],
                                  tsem).wait()

    lane = lax.broadcasted_iota(jnp.int32, (1, 128), 1)
    base = j * _TL

    def _tile_sum():
        placed = []
        for p in range(_TL):
            terms = []
            for c in range(8):
                off = pl.multiple_of(idxs[c, base + p], 128)
                w = wts[c, base + p]
                slab = tbl_vmem[:, pl.ds(off, 128)]
                wm = jnp.where(lane == lns[c, base + p], w, jnp.float32(0.0))
                terms.append(slab * wm)
            pacc = (((terms[0] + terms[1]) + (terms[2] + terms[3]))
                    + ((terms[4] + terms[5]) + (terms[6] + terms[7])))
            placed.append(jnp.sum(pacc, axis=1, keepdims=True))
        return jnp.concatenate(placed, axis=1)

    out_ref[...] = _tile_sum()


def kernel(x, table):
    b, d = x.shape
    t, f = table.shape
    assert d == 3 and f == 32
    assert t & (t - 1) == 0 and b % (2 * _TL) == 0

    x_t = jnp.transpose(x)
    tbl_t = jnp.transpose(table)
    pb = b // 2
    ntiles = pb // _TL
    kern = functools.partial(_ingp_kernel, tmask=t - 1)

    out_t = pl.pallas_call(
        kern,
        out_shape=jax.ShapeDtypeStruct((f, b), x.dtype),
        grid=(2, ntiles),
        in_specs=[
            pl.BlockSpec((3, pb), lambda i, j: (0, i)),
            pl.BlockSpec(memory_space=pl.ANY),
        ],
        out_specs=pl.BlockSpec((f, _TL), lambda i, j: (0, i * ntiles + j)),
        scratch_shapes=[
            pltpu.VMEM((f, t), jnp.float32),
            pltpu.VMEM((8, pb), jnp.int32),
            pltpu.VMEM((8, pb), jnp.int32),
            pltpu.VMEM((8, pb), jnp.float32),
            pltpu.SMEM((8, pb), jnp.int32),
            pltpu.SMEM((8, pb), jnp.int32),
            pltpu.SMEM((8, pb), jnp.float32),
            pltpu.SemaphoreType.DMA,
            pltpu.SemaphoreType.DMA,
        ],
        compiler_params=pltpu.CompilerParams(
            dimension_semantics=("parallel", "arbitrary"),
            vmem_limit_bytes=40 << 20,
            disable_bounds_checks=True,
        ),
    )(x_t, tbl_t)
    return jnp.transpose(out_t)
